# Initial kernel scaffold; baseline (speedup 1.0000x reference)
import jax
import jax.numpy as jnp
from jax.experimental import pallas as pl


def kernel(edge_index, edge_vals, input_feature, weight, bias):
    raise NotImplementedError("write your pallas kernel here")


# speedup vs baseline: 3.6476x; 3.6476x over previous
import jax
import jax.numpy as jnp
from jax import lax
from jax.experimental import pallas as pl
from jax.experimental.pallas import tpu as pltpu
from jax.experimental.pallas import tpu_sc as plsc

N_NODES = 10000
N_EDGES = 160000
D_IN = 256
D_OUT = 256
HALF = 128

NSUB = 16
C = 128
EPW = 10240
NCHUNK = EPW // C
E_PAD = NSUB * EPW
N_PAD = 10240
ROWS_PER_TILE = N_PAD // NSUB


def _sc_aggregate_body(x0, x1, cols, rows, vals, out0, out1,
                       col_v, rv_v, vv_v, gbuf, acc, gsem, rsem, vsem):
    c = lax.axis_index("c")
    s = lax.axis_index("s")

    def run(x_ref, out_ref):
        ebase = s * EPW
        pltpu.sync_copy(cols.at[pl.ds(ebase, EPW)], col_v)

        zero16 = jnp.zeros((16,), jnp.float32)

        def zrow(j, carry):
            for k in range(HALF // 16):
                gbuf[0, j, pl.ds(k * 16, 16)] = zero16
            return carry

        lax.fori_loop(0, C, zrow, 0)
        for i in range(ROWS_PER_TILE // C):
            pltpu.sync_copy(
                gbuf.at[0],
                acc.at[pl.ds(s * ROWS_PER_TILE + i * C, C)])
        plsc.subcore_barrier()

        def start_chunk(cur, b):
            off = ebase + cur * C
            pltpu.async_copy(
                x_ref.at[col_v.at[pl.ds(cur * C, C)]], gbuf.at[b],
                gsem.at[b])
            pltpu.async_copy(rows.at[pl.ds(off, C)], rv_v.at[b], rsem.at[b])
            pltpu.async_copy(vals.at[pl.ds(off, C)], vv_v.at[b], vsem.at[b])

        def wait_chunk(cur, b):
            pltpu.make_async_copy(
                rows.at[pl.ds(0, C)], rv_v.at[b], rsem.at[b]).wait()
            pltpu.make_async_copy(
                vals.at[pl.ds(0, C)], vv_v.at[b], vsem.at[b]).wait()
            pltpu.make_async_copy(
                x_ref.at[col_v.at[pl.ds(cur * C, C)]], gbuf.at[b],
                gsem.at[b]).wait()

        start_chunk(0, 0)

        def pair(p, carry):
            for b in (0, 1):
                cur = 2 * p + b
                wait_chunk(cur, b)
                nxt = cur + 1

                @pl.when(nxt < NCHUNK)
                def _():
                    start_chunk(nxt, 1 - b)

                def grp(g, carry2):
                    vals16 = vv_v[b, pl.ds(g * 16, 16)]
                    for j in range(16):
                        e = g * 16 + j
                        vb = jnp.full((16,), vals16[j])
                        for k in range(HALF // 16):
                            sl = pl.ds(k * 16, 16)
                            gbuf[b, e, sl] = gbuf[b, e, sl] * vb
                    return carry2

                lax.fori_loop(0, C // 16, grp, 0)

                pltpu.sync_copy(gbuf.at[b], acc.at[rv_v.at[b]], add=True)
            return carry

        lax.fori_loop(0, NCHUNK // 2, pair, 0)

        plsc.subcore_barrier()
        pltpu.sync_copy(
            acc.at[pl.ds(s * ROWS_PER_TILE, ROWS_PER_TILE)],
            out_ref.at[pl.ds(s * ROWS_PER_TILE, ROWS_PER_TILE)])

    @pl.when(c == 0)
    def _():
        run(x0, out0)

    @pl.when(c == 1)
    def _():
        run(x1, out1)


def _sc_aggregate(x0, x1, cols, rows, vals):
    mesh = plsc.VectorSubcoreMesh(core_axis_name="c", subcore_axis_name="s")
    kern = pl.kernel(
        _sc_aggregate_body,
        out_type=(
            jax.ShapeDtypeStruct((N_PAD, HALF), jnp.float32),
            jax.ShapeDtypeStruct((N_PAD, HALF), jnp.float32),
        ),
        mesh=mesh,
        scratch_types=[
            pltpu.VMEM((EPW,), jnp.int32),
            pltpu.VMEM((2, C), jnp.int32),
            pltpu.VMEM((2, C), jnp.float32),
            pltpu.VMEM((2, C, HALF), jnp.float32),
            pltpu.VMEM_SHARED((N_PAD, HALF), jnp.float32),
            pltpu.SemaphoreType.DMA((2,)),
            pltpu.SemaphoreType.DMA((2,)),
            pltpu.SemaphoreType.DMA((2,)),
        ],
    )
    return kern(x0, x1, cols, rows, vals)


def _mm_kernel(h0_ref, h1_ref, w0_ref, w1_ref, b_ref, o_ref):
    acc = jnp.dot(h0_ref[...], w0_ref[...],
                  preferred_element_type=jnp.float32)
    acc = acc + jnp.dot(h1_ref[...], w1_ref[...],
                        preferred_element_type=jnp.float32)
    o_ref[...] = acc + b_ref[...]


def _tc_matmul(h0, h1, w0, w1, bias2d):
    nblk = 10
    rows = N_PAD // nblk
    return pl.pallas_call(
        _mm_kernel,
        grid=(nblk,),
        in_specs=[
            pl.BlockSpec((rows, HALF), lambda i: (i, 0)),
            pl.BlockSpec((rows, HALF), lambda i: (i, 0)),
            pl.BlockSpec((HALF, D_OUT), lambda i: (0, 0)),
            pl.BlockSpec((HALF, D_OUT), lambda i: (0, 0)),
            pl.BlockSpec((1, D_OUT), lambda i: (0, 0)),
        ],
        out_specs=pl.BlockSpec((rows, D_OUT), lambda i: (i, 0)),
        out_shape=jax.ShapeDtypeStruct((N_PAD, D_OUT), jnp.float32),
    )(h0, h1, w0, w1, bias2d)


@jax.jit
def kernel(edge_index, edge_vals, input_feature, weight, bias):
    row = edge_index[0].astype(jnp.int32)
    col = edge_index[1].astype(jnp.int32)
    pad = E_PAD - N_EDGES
    col = jnp.pad(col, (0, pad))
    row = jnp.pad(row, (0, pad))
    vals = jnp.pad(edge_vals, (0, pad))

    x0 = input_feature[:, :HALF]
    x1 = input_feature[:, HALF:]

    h0, h1 = _sc_aggregate(x0, x1, col, row, vals)

    w0 = weight[:HALF]
    w1 = weight[HALF:]
    out = _tc_matmul(h0, h1, w0, w1, bias.reshape(1, D_OUT))
    return out[:N_NODES]
